# baseline (speedup 1.0000x reference)
import functools

import jax
import jax.numpy as jnp
from jax import lax
from jax.experimental import pallas as pl
from jax.experimental.pallas import tpu as pltpu
from jax.experimental.pallas import tpu_sc as plsc

VOCAB = 128
D = 64
BATCH = 4096
SEQ = 200
TOTAL = BATCH * SEQ
PAIRS = TOTAL // 2
PD = 2 * D
NC = 2
NS = 16
NW = NC * NS
PAIRS_PER_W = PAIRS // NW
CHUNK = 128
K = 2
STEP = CHUNK * K
N_STEPS = PAIRS_PER_W // STEP
IDX_ROWS_PER_W = PAIRS_PER_W // CHUNK


def _sc_gather(pidx2d, pair_table):
    mesh = plsc.VectorSubcoreMesh(core_axis_name="c", subcore_axis_name="s")

    @functools.partial(
        pl.kernel,
        mesh=mesh,
        out_type=jax.ShapeDtypeStruct((PAIRS, PD), jnp.float32),
        scratch_types=[
            pltpu.VMEM((IDX_ROWS_PER_W, CHUNK), jnp.int32),
            pltpu.VMEM((2, STEP, PD), jnp.float32),
            pltpu.SemaphoreType.DMA,
            pltpu.SemaphoreType.DMA,
        ],
        compiler_params=pltpu.CompilerParams(use_tc_tiling_on_sc=False),
    )
    def k(idx_hbm, ptab_hbm, out_hbm, idx_v, rows_v, sem_g, sem_o):
        cid = lax.axis_index("c")
        sid = lax.axis_index("s")
        wid = sid * NC + cid
        row0 = wid * IDX_ROWS_PER_W

        pltpu.sync_copy(idx_hbm.at[pl.ds(row0, IDX_ROWS_PER_W)], idx_v)

        def fire_gathers(i, slot):
            for j in range(K):
                pltpu.async_copy(
                    ptab_hbm.at[idx_v.at[i * K + j]],
                    rows_v.at[slot].at[pl.ds(j * CHUNK, CHUNK)],
                    sem_g,
                )

        def wait_gathers(slot):
            pltpu.make_async_copy(
                out_hbm.at[pl.ds(0, STEP)], rows_v.at[slot], sem_g
            ).wait()

        def fire_put(i, slot):
            pltpu.async_copy(
                rows_v.at[slot],
                out_hbm.at[pl.ds((row0 + i * K) * CHUNK, STEP)],
                sem_o,
            )

        def wait_put():
            pltpu.make_async_copy(
                rows_v.at[0], out_hbm.at[pl.ds(0, STEP)], sem_o
            ).wait()

        fire_gathers(0, 0)

        def step(i, carry):
            slot = lax.rem(i, 2)
            wait_gathers(slot)
            fire_put(i, slot)

            @pl.when(jnp.logical_and(i >= 1, i + 1 < N_STEPS))
            def _():
                wait_put()

            @pl.when(i + 1 < N_STEPS)
            def _():
                fire_gathers(i + 1, 1 - slot)

            return carry

        lax.fori_loop(0, N_STEPS, step, 0)
        wait_put()
        wait_put()

    return k(pidx2d, pair_table)


def kernel(smiles_indices, embedding_table):
    idx = smiles_indices.astype(jnp.int32).reshape(PAIRS, 2)
    pidx2d = (idx[:, 0] * VOCAB + idx[:, 1]).reshape(PAIRS // CHUNK, CHUNK)
    left = jnp.broadcast_to(embedding_table[:, None, :], (VOCAB, VOCAB, D))
    right = jnp.broadcast_to(embedding_table[None, :, :], (VOCAB, VOCAB, D))
    pair_table = jnp.concatenate([left, right], axis=-1).reshape(
        VOCAB * VOCAB, PD)
    out = _sc_gather(pidx2d, pair_table)
    return out.reshape(BATCH, SEQ, D)


# speedup vs baseline: 1.2972x; 1.2972x over previous
import functools

import jax
import jax.numpy as jnp
from jax import lax
from jax.experimental import pallas as pl
from jax.experimental.pallas import tpu as pltpu
from jax.experimental.pallas import tpu_sc as plsc

VOCAB = 128
D = 64
BATCH = 4096
SEQ = 200
HSEQ = SEQ // 2
TOTAL = BATCH * SEQ
PAIRS = TOTAL // 2
NC = 2
NS = 16
NW = NC * NS
PAIRS_PER_W = PAIRS // NW
CHUNK = 128
K = 2
STEP = CHUNK * K
N_STEPS = PAIRS_PER_W // STEP
IDX_ROWS_PER_W = PAIRS_PER_W // CHUNK


def _sc_gather(a2d, b2d, table):
    mesh = plsc.VectorSubcoreMesh(core_axis_name="c", subcore_axis_name="s")

    @functools.partial(
        pl.kernel,
        mesh=mesh,
        out_type=jax.ShapeDtypeStruct((PAIRS, 2 * D), jnp.float32),
        scratch_types=[
            pltpu.VMEM((IDX_ROWS_PER_W, CHUNK), jnp.int32),
            pltpu.VMEM((IDX_ROWS_PER_W, CHUNK), jnp.int32),
            pltpu.VMEM((2, 2, STEP, D), jnp.float32),
            pltpu.VMEM((VOCAB, D), jnp.float32),
            pltpu.VMEM_SHARED((VOCAB, D), jnp.float32),
            pltpu.SemaphoreType.DMA,
            pltpu.SemaphoreType.DMA,
        ],
        compiler_params=pltpu.CompilerParams(use_tc_tiling_on_sc=False),
    )
    def k(a_hbm, b_hbm, tab_hbm, out_hbm, a_v, b_v, rows_v, tab_v, tab_sh,
          sem_g, sem_o):
        cid = lax.axis_index("c")
        sid = lax.axis_index("s")
        wid = sid * NC + cid
        row0 = wid * IDX_ROWS_PER_W

        @pl.when(sid == 0)
        def _():
            pltpu.sync_copy(tab_hbm, tab_v)
            pltpu.sync_copy(tab_v, tab_sh)

        plsc.subcore_barrier()

        pltpu.sync_copy(a_hbm.at[pl.ds(row0, IDX_ROWS_PER_W)], a_v)
        pltpu.sync_copy(b_hbm.at[pl.ds(row0, IDX_ROWS_PER_W)], b_v)

        def fire_gathers(i, slot):
            for j in range(K):
                rs = pl.ds(j * CHUNK, CHUNK)
                pltpu.async_copy(
                    tab_sh.at[a_v.at[i * K + j]],
                    rows_v.at[slot].at[0].at[rs],
                    sem_g,
                )
                pltpu.async_copy(
                    tab_sh.at[b_v.at[i * K + j]],
                    rows_v.at[slot].at[1].at[rs],
                    sem_g,
                )

        def wait_gathers(slot):
            for j in range(K):
                rs = pl.ds(j * CHUNK, CHUNK)
                pltpu.make_async_copy(
                    tab_sh, rows_v.at[slot].at[0].at[rs], sem_g
                ).wait()
                pltpu.make_async_copy(
                    tab_sh, rows_v.at[slot].at[1].at[rs], sem_g
                ).wait()

        def fire_put(i, slot):
            o = pl.ds(wid * PAIRS_PER_W + i * STEP, STEP)
            pltpu.async_copy(
                rows_v.at[slot].at[0],
                out_hbm.at[o, pl.ds(0, D)],
                sem_o,
            )
            pltpu.async_copy(
                rows_v.at[slot].at[1],
                out_hbm.at[o, pl.ds(D, D)],
                sem_o,
            )

        def wait_put():
            pltpu.make_async_copy(
                rows_v.at[0].at[0], out_hbm.at[pl.ds(0, STEP), pl.ds(0, D)],
                sem_o,
            ).wait()
            pltpu.make_async_copy(
                rows_v.at[0].at[1], out_hbm.at[pl.ds(0, STEP), pl.ds(D, D)],
                sem_o,
            ).wait()

        fire_gathers(0, 0)

        def step(i, carry):
            slot = lax.rem(i, 2)
            wait_gathers(slot)
            fire_put(i, slot)

            @pl.when(jnp.logical_and(i >= 1, i + 1 < N_STEPS))
            def _():
                wait_put()

            @pl.when(i + 1 < N_STEPS)
            def _():
                fire_gathers(i + 1, 1 - slot)

            return carry

        lax.fori_loop(0, N_STEPS, step, 0)
        wait_put()
        wait_put()

    return k(a2d, b2d, table)


NB = 16


def _tc_unpack_body(in_ref, out_ref):
    x = in_ref[...].reshape(NB, HSEQ, 2 * D)
    out_ref[:, :HSEQ, :] = x[:, :, :D]
    out_ref[:, HSEQ:, :] = x[:, :, D:]


def _tc_unpack(mid):
    return pl.pallas_call(
        _tc_unpack_body,
        grid=(BATCH // NB,),
        in_specs=[pl.BlockSpec((NB * HSEQ, 2 * D), lambda g: (g, 0))],
        out_specs=pl.BlockSpec((NB, SEQ, D), lambda g: (g, 0, 0)),
        out_shape=jax.ShapeDtypeStruct((BATCH, SEQ, D), jnp.float32),
    )(mid)


def kernel(smiles_indices, embedding_table):
    idx = smiles_indices.astype(jnp.int32)
    a2d = idx[:, :HSEQ].reshape(PAIRS // CHUNK, CHUNK)
    b2d = idx[:, HSEQ:].reshape(PAIRS // CHUNK, CHUNK)
    mid = _sc_gather(a2d, b2d, embedding_table)
    return _tc_unpack(mid)
